# Initial kernel scaffold; baseline (speedup 1.0000x reference)
import jax
import jax.numpy as jnp
from jax.experimental import pallas as pl


def kernel(x, edge_index, W1, b1, W2, b2, W3, b3):
    raise NotImplementedError("write your pallas kernel here")


# speedup vs baseline: 14.8003x; 14.8003x over previous
import functools

import jax
import jax.numpy as jnp
from jax import lax
from jax.experimental import pallas as pl
from jax.experimental.pallas import tpu as pltpu
from jax.experimental.pallas import tpu_sc as plsc

N_NODES = 100000
N_EDGES = 3200000

NC = 2
NS = 16
NW = NC * NS

W_EDGES = 32
K_WIN = 16
CHUNK = W_EDGES * K_WIN
N_CHUNKS = 196
E_PW = CHUNK * N_CHUNKS
E_PAD = E_PW * NW

FW = 128
F = 16
NIX = W_EDGES * F // 128

RPT = 6272
N_ZERO = RPT * NS
N_ACC16 = N_ZERO * F
JUNK = N_ZERO - N_NODES
ZCH = 3584

ROW_BLK = 2000
N_BLKS = N_NODES // ROW_BLK

_MESH = plsc.VectorSubcoreMesh(
    core_axis_name="c", subcore_axis_name="s", num_cores=NC, num_subcores=NS
)


def _zero_vmem_1d(ref, n):
    def body(i, carry):
        ref[pl.ds(i * 16, 16)] = jnp.zeros((16,), jnp.float32)
        return carry
    lax.fori_loop(0, n // 16, body, 0)


def _deg_body(coli, out, acc, col_v, ones_v, zbuf, sem_s):
    c = lax.axis_index("c")
    s = lax.axis_index("s")
    w = s * NC + c
    r0 = s * RPT
    def ones_body(i, carry):
        ones_v[pl.ds(i * 16, 16)] = jnp.ones((16,), jnp.float32)
        return carry
    lax.fori_loop(0, W_EDGES // 16, ones_body, 0)
    _zero_vmem_1d(zbuf, RPT)
    pltpu.sync_copy(zbuf, acc.at[pl.ds(r0, RPT)])
    plsc.subcore_barrier()

    def chunk(ci, carry):
        pltpu.sync_copy(coli.at[w, ci], col_v)
        for j in range(K_WIN):
            pltpu.async_copy(ones_v, acc.at[col_v.at[j]], sem_s, add=True)
        for j in range(K_WIN):
            pltpu.make_async_copy(ones_v, acc.at[col_v.at[j]], sem_s).wait()
        return carry

    lax.fori_loop(0, N_CHUNKS, chunk, 0)
    plsc.subcore_barrier()
    pltpu.sync_copy(acc.at[pl.ds(r0, RPT)], zbuf)
    pltpu.sync_copy(zbuf, out.at[pl.ds(c * N_ZERO + r0, RPT)])


_deg_call = pl.kernel(
    _deg_body,
    out_type=jax.ShapeDtypeStruct((NC * N_ZERO,), jnp.float32),
    mesh=_MESH,
    scratch_types=[
        pltpu.VMEM_SHARED((N_ZERO,), jnp.float32),
        pltpu.VMEM((K_WIN, W_EDGES), jnp.int32),
        pltpu.VMEM((W_EDGES,), jnp.float32),
        pltpu.VMEM((RPT,), jnp.float32),
        pltpu.SemaphoreType.DMA,
    ],
)


def _agg_body(
    g, rowi, ixf, out, acc, row_v, ixv, msgs, sm0, sm1, zbuf, sem_g, sem_s
):
    smsg = [sm0, sm1]
    c = lax.axis_index("c")
    s = lax.axis_index("s")
    w = s * NC + c
    rz = s * RPT * F
    _zero_vmem_1d(zbuf, ZCH)

    def zinit(q, carry):
        pltpu.sync_copy(zbuf, acc.at[pl.ds(rz + q * ZCH, ZCH)])
        return carry

    lax.fori_loop(0, RPT * F // ZCH, zinit, 0)
    plsc.subcore_barrier()

    def chunk(ci, carry):
        pltpu.sync_copy(rowi.at[w, ci], row_v)
        pltpu.sync_copy(ixf.at[w, ci], ixv)
        for j in range(2):
            pltpu.async_copy(g.at[row_v.at[j]], msgs.at[j], sem_g)
        for j in range(K_WIN):
            slot = j % 2
            if j >= 2:
                for k in range(NIX):
                    pltpu.make_async_copy(
                        smsg[slot].at[pl.ds(k * 128, 128)],
                        acc.at[ixv.at[j - 2, k]],
                        sem_s,
                    ).wait()
            pltpu.make_async_copy(
                g.at[row_v.at[j]], msgs.at[slot], sem_g
            ).wait()

            def compact(e, carry):
                for u in range(8):
                    smsg[slot][pl.ds((e * 8 + u) * F, F)] = msgs[
                        slot, e * 8 + u, pl.ds(0, F)
                    ]
                return carry

            lax.fori_loop(0, W_EDGES // 8, compact, 0)
            if j + 2 < K_WIN:
                pltpu.async_copy(g.at[row_v.at[j + 2]], msgs.at[slot], sem_g)
            for k in range(NIX):
                pltpu.async_copy(
                    smsg[slot].at[pl.ds(k * 128, 128)],
                    acc.at[ixv.at[j, k]],
                    sem_s,
                    add=True,
                )
        for j in range(K_WIN - 2, K_WIN):
            for k in range(NIX):
                pltpu.make_async_copy(
                    smsg[j % 2].at[pl.ds(k * 128, 128)],
                    acc.at[ixv.at[j, k]],
                    sem_s,
                ).wait()
        return carry

    lax.fori_loop(0, N_CHUNKS, chunk, 0)
    plsc.subcore_barrier()

    def cpout(q, carry):
        pltpu.sync_copy(acc.at[pl.ds(rz + q * ZCH, ZCH)], zbuf)
        pltpu.sync_copy(
            zbuf, out.at[pl.ds(c * N_ZERO * F + rz + q * ZCH, ZCH)]
        )
        return carry

    lax.fori_loop(0, RPT * F // ZCH, cpout, 0)


_agg_call = pl.kernel(
    _agg_body,
    out_type=jax.ShapeDtypeStruct((NC * N_ZERO * F,), jnp.float32),
    mesh=_MESH,
    scratch_types=[
        pltpu.VMEM_SHARED((N_ACC16,), jnp.float32),
        pltpu.VMEM((K_WIN, W_EDGES), jnp.int32),
        pltpu.VMEM((K_WIN, NIX, 128), jnp.int32),
        pltpu.VMEM((2, W_EDGES, FW), jnp.float32),
        pltpu.VMEM((W_EDGES * F,), jnp.float32),
        pltpu.VMEM((W_EDGES * F,), jnp.float32),
        pltpu.VMEM((ZCH,), jnp.float32),
        pltpu.SemaphoreType.DMA,
        pltpu.SemaphoreType.DMA,
    ],
)


def _prep_body(deg_ref, x_ref, dinv_ref, g1_ref):
    deg = deg_ref[0] + deg_ref[1] + 1.0
    dinv = lax.rsqrt(deg)
    dinv_ref[...] = dinv
    g1 = x_ref[...] * dinv
    g1_ref[...] = jnp.concatenate(
        [g1, jnp.zeros((g1.shape[0], FW - 2), jnp.float32)], axis=1
    )


def _prep_call(deg2, x):
    return pl.pallas_call(
        _prep_body,
        grid=(N_BLKS,),
        in_specs=[
            pl.BlockSpec((NC, ROW_BLK, 1), lambda i: (0, i, 0)),
            pl.BlockSpec((ROW_BLK, 2), lambda i: (i, 0)),
        ],
        out_specs=[
            pl.BlockSpec((ROW_BLK, 1), lambda i: (i, 0)),
            pl.BlockSpec((ROW_BLK, FW), lambda i: (i, 0)),
        ],
        out_shape=[
            jax.ShapeDtypeStruct((N_NODES, 1), jnp.float32),
            jax.ShapeDtypeStruct((N_NODES, FW), jnp.float32),
        ],
    )(deg2, x)


def _layer_body(f_use, relu, a_ref, g_ref, dinv_ref, w_ref, b_ref, out_ref):
    dinv = dinv_ref[...]
    agg = (a_ref[0] + a_ref[1] + g_ref[:, :F]) * dinv
    h = (
        jnp.dot(agg[:, :f_use], w_ref[...], preferred_element_type=jnp.float32)
        + b_ref[...]
    )
    if relu:
        h = jnp.maximum(h, 0.0)
        h = h * dinv
        h = jnp.concatenate(
            [h, jnp.zeros((h.shape[0], FW - F), jnp.float32)], axis=1
        )
    out_ref[...] = h


def _layer_call(a, g, dinv, w, b, f_use, f_out, relu):
    fo = FW if relu else f_out
    return pl.pallas_call(
        functools.partial(_layer_body, f_use, relu),
        grid=(N_BLKS,),
        in_specs=[
            pl.BlockSpec((NC, ROW_BLK, F), lambda i: (0, i, 0)),
            pl.BlockSpec((ROW_BLK, FW), lambda i: (i, 0)),
            pl.BlockSpec((ROW_BLK, 1), lambda i: (i, 0)),
            pl.BlockSpec((f_use, f_out), lambda i: (0, 0)),
            pl.BlockSpec((1, f_out), lambda i: (0, 0)),
        ],
        out_specs=pl.BlockSpec((ROW_BLK, fo), lambda i: (i, 0)),
        out_shape=jax.ShapeDtypeStruct((N_NODES, fo), jnp.float32),
    )(a, g, dinv, w, b.reshape(1, f_out))


def kernel(x, edge_index, W1, b1, W2, b2, W3, b3):
    row = edge_index[0].astype(jnp.int32)
    col = edge_index[1].astype(jnp.int32)
    pad = E_PAD - N_EDGES
    pad_ar = lax.iota(jnp.int32, pad)
    row_p = jnp.concatenate([row, pad_ar % N_NODES])
    col_p = jnp.concatenate([col, N_NODES + pad_ar % JUNK])
    rowi = row_p.reshape(NW, N_CHUNKS, K_WIN, W_EDGES)
    coli = col_p.reshape(NW, N_CHUNKS, K_WIN, W_EDGES)
    ixf = (col_p[:, None] * F + lax.iota(jnp.int32, F)[None, :]).reshape(
        NW, N_CHUNKS, K_WIN, NIX, 128
    )

    deg2 = _deg_call(coli).reshape(NC, N_ZERO, 1)
    dinv, g1 = _prep_call(deg2, x)
    a1 = _agg_call(g1, rowi, ixf).reshape(NC, N_ZERO, F)
    g2 = _layer_call(a1, g1, dinv, W1, b1, 2, 16, True)
    a2 = _agg_call(g2, rowi, ixf).reshape(NC, N_ZERO, F)
    g3 = _layer_call(a2, g2, dinv, W2, b2, 16, 16, True)
    a3 = _agg_call(g3, rowi, ixf).reshape(NC, N_ZERO, F)
    out = _layer_call(a3, g3, dinv, W3, b3, 16, 64, False)
    return out
